# baseline (speedup 1.0000x reference)
import jax
import jax.numpy as jnp
from jax import lax
from jax.experimental import pallas as pl
from jax.experimental.pallas import tpu as pltpu, tpu_sc as plsc

B = 16384
EMBED = 64
SUB = 8
NC, NS, L = 2, 16, 16
NW = NC * NS
BPW = B // NW
NCHUNK = BPW // L


def _rsqrt(x):
    i = plsc.bitcast(x, jnp.int32)
    i = jnp.int32(0x5F3759DF) - (i >> 1)
    y = plsc.bitcast(i, jnp.float32)
    for _ in range(3):
        y = y * (1.5 - 0.5 * x * y * y)
    return y


def _body(fidx, tidx, ftab, ttab, wvec_h, bvec_h, out_h,
          idx_fv, idx_tv, bf0, bf1, bf2, bf3, bt0, bt1,
          out_v, wb_v, sf0, sf1, sf2, sf3, st0, st1):
    wid = lax.axis_index("s") * NC + lax.axis_index("c")
    base = wid * BPW

    pltpu.sync_copy(fidx.at[pl.ds(base, BPW)], idx_fv)
    pltpu.sync_copy(tidx.at[pl.ds(base, BPW)], idx_tv)
    pltpu.sync_copy(wvec_h, wb_v.at[0])
    pltpu.sync_copy(bvec_h, wb_v.at[1])

    w = wb_v[0]
    b = wb_v[1]
    eps = jnp.full((L,), 1e-12, jnp.float32)
    lane = lax.iota(jnp.int32, L)
    zero = jnp.zeros((L,), jnp.float32)

    def fire_f(c, bf, sf):
        tv_f = idx_fv[pl.ds(c * L, L)] >> 3
        for k in range(L):
            pltpu.async_copy(ftab.at[tv_f[k]], bf.at[k], sf)

    def fire_t(c, bt, st):
        tv_t = idx_tv[pl.ds(c * L, L)] >> 3
        for k in range(L):
            pltpu.async_copy(ttab.at[tv_t[k]], bt.at[k], st)

    def drain_f(bf, sf):
        pltpu.make_async_copy(ftab.at[pl.ds(0, L)], bf, sf).wait()

    def drain_t(bt, st):
        pltpu.make_async_copy(ttab.at[pl.ds(0, L)], bt, st).wait()

    def compute(c, bf, bt):
        iv_f = idx_fv[pl.ds(c * L, L)]
        iv_t = idx_tv[pl.ds(c * L, L)]
        sub_f = iv_f & 7
        sub_t = iv_t & 7

        def body(d, carry):
            acc_d, acc_a, acc_b = carry
            col = jnp.full((L,), d, jnp.int32)
            gf = plsc.load_gather(bf, [lane, sub_f, col])
            gt = plsc.load_gather(bt, [lane, sub_t, col])
            return (acc_d + gf * gt, acc_a + gf * gf, acc_b + gt * gt)

        acc_d, acc_a, acc_b = lax.fori_loop(
            0, EMBED, body, (zero, zero, zero), unroll=16)

        inv = _rsqrt(jnp.maximum(acc_a, eps) * jnp.maximum(acc_b, eps))
        z = acc_d * inv * w + b
        out_v[pl.ds(c * L, L)] = 1.0 / (1.0 + jnp.exp(-z))

    fbufs = ((bf0, sf0), (bf1, sf1), (bf2, sf2), (bf3, sf3))
    tbufs = ((bt0, st0), (bt1, st1))
    for j in range(3):
        fire_f(j, *fbufs[j])
    for j in range(2):
        fire_t(j, *tbufs[j])

    def step(i, _):
        for j in range(4):
            c = 4 * i + j
            drain_f(*fbufs[j])
            drain_t(*tbufs[j % 2])
            compute(c, fbufs[j][0], tbufs[j % 2][0])

            @pl.when(c + 2 < NCHUNK)
            def _():
                fire_t(c + 2, *tbufs[j % 2])

            @pl.when(c + 3 < NCHUNK)
            def _():
                fire_f(c + 3, *fbufs[(j + 3) % 4])
        return 0

    lax.fori_loop(0, NCHUNK // 4, step, 0)
    pltpu.sync_copy(out_v, out_h.at[pl.ds(base, BPW)])


@jax.jit
def kernel(file, test, file_table, test_table, dense_w, dense_b):
    fidx = file.astype(jnp.int32)
    tidx = test.astype(jnp.int32)
    ft3 = file_table.reshape(file_table.shape[0] // SUB, SUB, EMBED)
    tt3 = test_table.reshape(test_table.shape[0] // SUB, SUB, EMBED)
    wvec = jnp.broadcast_to(dense_w.reshape(1), (L,)).astype(jnp.float32)
    bvec = jnp.broadcast_to(dense_b.reshape(1), (L,)).astype(jnp.float32)

    mesh = plsc.VectorSubcoreMesh(core_axis_name="c", subcore_axis_name="s")
    out = pl.kernel(
        _body,
        out_type=jax.ShapeDtypeStruct((B,), jnp.float32),
        mesh=mesh,
        compiler_params=pltpu.CompilerParams(needs_layout_passes=False),
        scratch_types=[
            pltpu.VMEM((BPW,), jnp.int32),
            pltpu.VMEM((BPW,), jnp.int32),
            pltpu.VMEM((L, SUB, EMBED), jnp.float32),
            pltpu.VMEM((L, SUB, EMBED), jnp.float32),
            pltpu.VMEM((L, SUB, EMBED), jnp.float32),
            pltpu.VMEM((L, SUB, EMBED), jnp.float32),
            pltpu.VMEM((L, SUB, EMBED), jnp.float32),
            pltpu.VMEM((L, SUB, EMBED), jnp.float32),
            pltpu.VMEM((BPW,), jnp.float32),
            pltpu.VMEM((2, L), jnp.float32),
            pltpu.SemaphoreType.DMA,
            pltpu.SemaphoreType.DMA,
            pltpu.SemaphoreType.DMA,
            pltpu.SemaphoreType.DMA,
            pltpu.SemaphoreType.DMA,
            pltpu.SemaphoreType.DMA,
        ],
    )(fidx, tidx, ft3, tt3, wvec, bvec)
    return out.reshape(B, 1)


# speedup vs baseline: 1.0070x; 1.0070x over previous
import jax
import jax.numpy as jnp
from jax import lax
from jax.experimental import pallas as pl
from jax.experimental.pallas import tpu as pltpu, tpu_sc as plsc

B = 16384
EMBED = 64
SUB = 8
NC, NS, L = 2, 16, 16
NW = NC * NS
BPW = B // NW
NCHUNK = BPW // L


def _rsqrt(x):
    i = plsc.bitcast(x, jnp.int32)
    i = jnp.int32(0x5F3759DF) - (i >> 1)
    y = plsc.bitcast(i, jnp.float32)
    for _ in range(3):
        y = y * (1.5 - 0.5 * x * y * y)
    return y


def _body(fidx, tidx, ftab, ttab, wvec_h, bvec_h, out_h,
          idx_fv, idx_tv, bf0, bt0, bf1, bt1, out_v, wb_v,
          sf0, st0, sf1, st1):
    wid = lax.axis_index("s") * NC + lax.axis_index("c")
    base = wid * BPW

    pltpu.sync_copy(fidx.at[pl.ds(base, BPW)], idx_fv)
    pltpu.sync_copy(tidx.at[pl.ds(base, BPW)], idx_tv)
    pltpu.sync_copy(wvec_h, wb_v.at[0])
    pltpu.sync_copy(bvec_h, wb_v.at[1])

    w = wb_v[0]
    b = wb_v[1]
    eps = jnp.full((L,), 1e-12, jnp.float32)
    lane = lax.iota(jnp.int32, L)
    zero = jnp.zeros((L,), jnp.float32)

    def fire(c, bf, bt, sf, st):
        tv_f = idx_fv[pl.ds(c * L, L)] >> 3
        tv_t = idx_tv[pl.ds(c * L, L)] >> 3
        for k in range(L):
            pltpu.async_copy(ftab.at[tv_f[k]], bf.at[k], sf)
            pltpu.async_copy(ttab.at[tv_t[k]], bt.at[k], st)

    def drain(bf, bt, sf, st):
        pltpu.make_async_copy(ftab.at[pl.ds(0, L)], bf, sf).wait()
        pltpu.make_async_copy(ttab.at[pl.ds(0, L)], bt, st).wait()

    def compute(c, bf, bt):
        iv_f = idx_fv[pl.ds(c * L, L)]
        iv_t = idx_tv[pl.ds(c * L, L)]
        sub_f = iv_f & 7
        sub_t = iv_t & 7

        def body(d, carry):
            acc_d, acc_a, acc_b = carry
            col = jnp.full((L,), d, jnp.int32)
            gf = plsc.load_gather(bf, [lane, sub_f, col])
            gt = plsc.load_gather(bt, [lane, sub_t, col])
            return (acc_d + gf * gt, acc_a + gf * gf, acc_b + gt * gt)

        acc_d, acc_a, acc_b = lax.fori_loop(
            0, EMBED, body, (zero, zero, zero), unroll=16)

        inv = _rsqrt(jnp.maximum(acc_a, eps) * jnp.maximum(acc_b, eps))
        z = acc_d * inv * w + b
        out_v[pl.ds(c * L, L)] = 1.0 / (1.0 + jnp.exp(-z))

    fire(0, bf0, bt0, sf0, st0)

    def step(i, _):
        c0 = 2 * i
        fire(c0 + 1, bf1, bt1, sf1, st1)
        drain(bf0, bt0, sf0, st0)
        compute(c0, bf0, bt0)

        @pl.when(c0 + 2 < NCHUNK)
        def _():
            fire(c0 + 2, bf0, bt0, sf0, st0)

        drain(bf1, bt1, sf1, st1)
        compute(c0 + 1, bf1, bt1)
        return 0

    lax.fori_loop(0, NCHUNK // 2, step, 0)
    pltpu.sync_copy(out_v, out_h.at[pl.ds(base, BPW)])


@jax.jit
def kernel(file, test, file_table, test_table, dense_w, dense_b):
    fidx = file.astype(jnp.int32)
    tidx = test.astype(jnp.int32)
    ft3 = file_table.reshape(file_table.shape[0] // SUB, SUB, EMBED)
    tt3 = test_table.reshape(test_table.shape[0] // SUB, SUB, EMBED)
    wvec = jnp.broadcast_to(dense_w.reshape(1), (L,)).astype(jnp.float32)
    bvec = jnp.broadcast_to(dense_b.reshape(1), (L,)).astype(jnp.float32)

    mesh = plsc.VectorSubcoreMesh(core_axis_name="c", subcore_axis_name="s")
    out = pl.kernel(
        _body,
        out_type=jax.ShapeDtypeStruct((B,), jnp.float32),
        mesh=mesh,
        compiler_params=pltpu.CompilerParams(needs_layout_passes=False),
        scratch_types=[
            pltpu.VMEM((BPW,), jnp.int32),
            pltpu.VMEM((BPW,), jnp.int32),
            pltpu.VMEM((L, SUB, EMBED), jnp.float32),
            pltpu.VMEM((L, SUB, EMBED), jnp.float32),
            pltpu.VMEM((L, SUB, EMBED), jnp.float32),
            pltpu.VMEM((L, SUB, EMBED), jnp.float32),
            pltpu.VMEM((BPW,), jnp.float32),
            pltpu.VMEM((2, L), jnp.float32),
            pltpu.SemaphoreType.DMA,
            pltpu.SemaphoreType.DMA,
            pltpu.SemaphoreType.DMA,
            pltpu.SemaphoreType.DMA,
        ],
    )(fidx, tidx, ft3, tt3, wvec, bvec)
    return out.reshape(B, 1)
